# Initial kernel scaffold; baseline (speedup 1.0000x reference)
import jax
import jax.numpy as jnp
from jax.experimental import pallas as pl


def kernel(body_verts, verts, faces):
    raise NotImplementedError("write your pallas kernel here")


# speedup vs baseline: 4.7000x; 4.7000x over previous
import functools

import jax
import jax.numpy as jnp
from jax.experimental import pallas as pl
from jax.experimental.pallas import tpu as pltpu
from jax.experimental.pallas import tpu_sc as plsc

F = 4096
V = 8192
N = 16384
NC, NS = 2, 16
NW = NC * NS
FPW = F // NW
ROWS = 40


P_BLK = 512
F_BLK = 1024
NP = N // P_BLK
NF = F // F_BLK


def _sc_face_setup(vxh, vyh, vzh, fah, fbh, fch, out,
                   vx, vy, vz, fa, fb, fc, stage):
    wid = jax.lax.axis_index("s") * NC + jax.lax.axis_index("c")
    base = wid * FPW
    pltpu.sync_copy(vxh, vx)
    pltpu.sync_copy(vyh, vy)
    pltpu.sync_copy(vzh, vz)
    pltpu.sync_copy(fah.at[pl.ds(base, FPW)], fa)
    pltpu.sync_copy(fbh.at[pl.ds(base, FPW)], fb)
    pltpu.sync_copy(fch.at[pl.ds(base, FPW)], fc)

    zero = jnp.zeros((16,), jnp.float32)
    for j in range(FPW // 16):
        sl = pl.ds(j * 16, 16)
        ia = fa[sl]
        ib = fb[sl]
        ic = fc[sl]
        ax = plsc.load_gather(vx, [ia])
        ay = plsc.load_gather(vy, [ia])
        az = plsc.load_gather(vz, [ia])
        bx = plsc.load_gather(vx, [ib])
        by = plsc.load_gather(vy, [ib])
        bz = plsc.load_gather(vz, [ib])
        cx = plsc.load_gather(vx, [ic])
        cy = plsc.load_gather(vy, [ic])
        cz = plsc.load_gather(vz, [ic])
        abx, aby, abz = bx - ax, by - ay, bz - az
        acx, acy, acz = cx - ax, cy - ay, cz - az
        cbx, cby, cbz = cx - bx, cy - by, cz - bz
        kab_a = abx * ax + aby * ay + abz * az
        kab_b = abx * bx + aby * by + abz * bz
        kab_c = abx * cx + aby * cy + abz * cz
        kac_a = acx * ax + acy * ay + acz * az
        kac_b = acx * bx + acy * by + acz * bz
        kac_c = acx * cx + acy * cy + acz * cz
        na2 = ax * ax + ay * ay + az * az
        lab = abx * abx + aby * aby + abz * abz
        lac = acx * acx + acy * acy + acz * acz
        lbc = cbx * cbx + cby * cby + cbz * cbz
        m = abx * acx + aby * acy + abz * acz
        one = jnp.ones((16,), jnp.float32)
        rab = one / jnp.where(lab == 0.0, one, lab)
        rac = one / jnp.where(lac == 0.0, one, lac)
        rbc = one / jnp.where(lbc == 0.0, one, lbc)
        stage[0, sl] = abx
        stage[1, sl] = aby
        stage[2, sl] = abz
        stage[8, sl] = acx
        stage[9, sl] = acy
        stage[10, sl] = acz
        stage[16, sl] = -2.0 * ax
        stage[17, sl] = -2.0 * ay
        stage[18, sl] = -2.0 * az
        stage[24, sl] = kab_a
        stage[25, sl] = kab_b
        stage[26, sl] = kab_c
        stage[27, sl] = kac_a
        stage[28, sl] = kac_b
        stage[29, sl] = kac_c
        stage[30, sl] = na2
        stage[31, sl] = lab
        stage[32, sl] = lac
        stage[33, sl] = rab
        stage[34, sl] = rac
        stage[35, sl] = rbc
        stage[36, sl] = lbc
        stage[37, sl] = m
        for r in (3, 4, 5, 6, 7, 11, 12, 13, 14, 15, 19, 20, 21, 22, 23, 38, 39):
            stage[r, sl] = zero
    pltpu.sync_copy(stage, out.at[:, pl.ds(base, FPW)])


@jax.jit
def _face_setup(vxh, vyh, vzh, fah, fbh, fch):
    kfn = pl.kernel(
        _sc_face_setup,
        out_type=jax.ShapeDtypeStruct((ROWS, F), jnp.float32),
        mesh=plsc.VectorSubcoreMesh(core_axis_name="c", subcore_axis_name="s"),
        scratch_types=[
            pltpu.VMEM((V,), jnp.float32),
            pltpu.VMEM((V,), jnp.float32),
            pltpu.VMEM((V,), jnp.float32),
            pltpu.VMEM((FPW,), jnp.int32),
            pltpu.VMEM((FPW,), jnp.int32),
            pltpu.VMEM((FPW,), jnp.int32),
            pltpu.VMEM((ROWS, FPW), jnp.float32),
        ],
        compiler_params=pltpu.CompilerParams(needs_layout_passes=False),
    )
    return kfn(vxh, vyh, vzh, fah, fbh, fch)


def _tc_dist(pts_ref, fc_ref, loss_ref, acc_ref):
    i = pl.program_id(0)
    j = pl.program_id(1)

    pts = pts_ref[...]
    dot = functools.partial(
        jax.lax.dot_general,
        dimension_numbers=(((1,), (0,)), ((), ())),
        preferred_element_type=jnp.float32,
        precision=jax.lax.Precision.HIGHEST,
    )
    g1 = dot(pts, fc_ref[0:8, :])
    g2 = dot(pts, fc_ref[8:16, :])
    g3 = dot(pts, fc_ref[16:24, :])

    row = lambda r: fc_ref[r:r + 1, :]
    pp = jnp.sum(pts * pts, axis=1, keepdims=True)

    d1 = g1 - row(24)
    d3 = g1 - row(25)
    d5 = g1 - row(26)
    d2 = g2 - row(27)
    d4 = g2 - row(28)
    d6 = g2 - row(29)
    ap2 = pp + g3 + row(30)
    lab = row(31)
    lac = row(32)
    lbc = row(36)

    va = d3 * d6 - d5 * d4
    vb = d5 * d2 - d1 * d6
    vc = d1 * d4 - d3 * d2
    den = va + vb + vc
    inside = (va >= 0.0) & (vb >= 0.0) & (vc >= 0.0) & (den > 0.0)
    rden = 1.0 / jnp.where(den == 0.0, 1.0, den)
    v = vb * rden
    w = vc * rden
    df = (ap2 - 2.0 * (v * d1 + w * d2)
          + (v * v) * lab + (w * w) * lac + (2.0 * v * w) * row(37))
    df = jnp.where(inside, df, 1e30)

    d1t = d1 + d1
    ta = jnp.clip(d1 * row(33), 0.0, 1.0)
    dab = ap2 - ta * (d1t - ta * lab)
    tc = jnp.clip(d2 * row(34), 0.0, 1.0)
    dac = ap2 - tc * (d2 + d2 - tc * lac)
    e = d4 - d3
    bp2 = ap2 - d1t + lab
    tb = jnp.clip(e * row(35), 0.0, 1.0)
    dbc = bp2 - tb * (e + e - tb * lbc)

    d = jnp.minimum(jnp.minimum(dab, dac), jnp.minimum(dbc, df))
    dmin = jnp.min(d, axis=1, keepdims=True)

    @pl.when((i == 0) & (j == 0))
    def _():
        loss_ref[0, 0] = 0.0

    @pl.when(j == 0)
    def _():
        acc_ref[...] = dmin

    @pl.when(j > 0)
    def _():
        acc_ref[...] = jnp.minimum(acc_ref[...], dmin)

    @pl.when(j == NF - 1)
    def _():
        loss_ref[0, 0] += jnp.sum(acc_ref[...]) * (1.0 / N)


@functools.partial(jax.jit, static_argnames=("interpret",))
def _point_mesh_loss(pts8, fc, interpret=False):
    out = pl.pallas_call(
        _tc_dist,
        grid=(NP, NF),
        in_specs=[
            pl.BlockSpec((P_BLK, 8), lambda i, j: (i, 0)),
            pl.BlockSpec((ROWS, F_BLK), lambda i, j: (0, j)),
        ],
        out_specs=pl.BlockSpec(memory_space=pltpu.SMEM),
        out_shape=jax.ShapeDtypeStruct((1, 1), jnp.float32),
        scratch_shapes=[pltpu.VMEM((P_BLK, 1), jnp.float32)],
        compiler_params=pltpu.CompilerParams(
            dimension_semantics=("arbitrary", "arbitrary"),
        ),
        interpret=interpret,
    )(pts8, fc)
    return out[0, 0]


def kernel(body_verts, verts, faces):
    fi = faces.astype(jnp.int32)
    fc = _face_setup(verts[:, 0], verts[:, 1], verts[:, 2],
                     fi[:, 0], fi[:, 1], fi[:, 2])
    pts = body_verts.reshape(-1, 3)
    pts8 = jnp.concatenate([pts, jnp.zeros((N, 5), jnp.float32)], axis=1)
    return _point_mesh_loss(pts8, fc)


# speedup vs baseline: 5.0284x; 1.0699x over previous
import functools

import jax
import jax.numpy as jnp
from jax.experimental import pallas as pl
from jax.experimental.pallas import tpu as pltpu
from jax.experimental.pallas import tpu_sc as plsc

F = 4096
V = 8192
N = 16384
NC, NS = 2, 16
NW = NC * NS
FPW = F // NW
FCR = 16

P_BLK = 512
F_BLK = 1024
NP = N // P_BLK
NF = F // F_BLK

R_LAB, R_LAC, R_LBC, R_RAB, R_RAC, R_RBC, R_RD, R_EBC, R_DG, R_D, R_M = range(11)


def _sc_face_setup(vxh, vyh, vzh, fah, fbh, fch, rhs, fcs,
                   vx, vy, vz, fa, fb, fc, s0, s1, s2, s3, sfc):
    wid = jax.lax.axis_index("s") * NC + jax.lax.axis_index("c")
    base = wid * FPW
    pltpu.sync_copy(vxh, vx)
    pltpu.sync_copy(vyh, vy)
    pltpu.sync_copy(vzh, vz)
    pltpu.sync_copy(fah.at[pl.ds(base, FPW)], fa)
    pltpu.sync_copy(fbh.at[pl.ds(base, FPW)], fb)
    pltpu.sync_copy(fch.at[pl.ds(base, FPW)], fc)

    zero = jnp.zeros((16,), jnp.float32)
    one = jnp.ones((16,), jnp.float32)
    for j in range(FPW // 16):
        sl = pl.ds(j * 16, 16)
        ia = fa[sl]
        ib = fb[sl]
        ic = fc[sl]
        ax = plsc.load_gather(vx, [ia])
        ay = plsc.load_gather(vy, [ia])
        az = plsc.load_gather(vz, [ia])
        bx = plsc.load_gather(vx, [ib])
        by = plsc.load_gather(vy, [ib])
        bz = plsc.load_gather(vz, [ib])
        cx = plsc.load_gather(vx, [ic])
        cy = plsc.load_gather(vy, [ic])
        cz = plsc.load_gather(vz, [ic])
        abx, aby, abz = bx - ax, by - ay, bz - az
        acx, acy, acz = cx - ax, cy - ay, cz - az
        cbx, cby, cbz = cx - bx, cy - by, cz - bz
        nx = aby * acz - abz * acy
        ny = abz * acx - abx * acz
        nz = abx * acy - aby * acx
        kab_a = abx * ax + aby * ay + abz * az
        kac_a = acx * ax + acy * ay + acz * az
        n_a = nx * ax + ny * ay + nz * az
        na2 = ax * ax + ay * ay + az * az
        lab = abx * abx + aby * aby + abz * abz
        lac = acx * acx + acy * acy + acz * acz
        lbc = cbx * cbx + cby * cby + cbz * cbz
        m = abx * acx + aby * acy + abz * acz
        dd = nx * nx + ny * ny + nz * nz
        rab = one / jnp.where(lab == 0.0, one, lab)
        rac = one / jnp.where(lac == 0.0, one, lac)
        rbc = one / jnp.where(lbc == 0.0, one, lbc)
        rd = one / jnp.where(dd == 0.0, one, dd)
        dgate = jnp.where(dd > 1e-6 * (lab * lac), zero, jnp.full((16,), 1e30, jnp.float32))

        s0[0, sl] = abx
        s0[1, sl] = aby
        s0[2, sl] = abz
        s0[3, sl] = zero
        s0[4, sl] = -kab_a
        s1[0, sl] = acx
        s1[1, sl] = acy
        s1[2, sl] = acz
        s1[3, sl] = zero
        s1[4, sl] = -kac_a
        s2[0, sl] = -2.0 * ax
        s2[1, sl] = -2.0 * ay
        s2[2, sl] = -2.0 * az
        s2[3, sl] = one
        s2[4, sl] = na2
        s3[0, sl] = nx
        s3[1, sl] = ny
        s3[2, sl] = nz
        s3[3, sl] = zero
        s3[4, sl] = -n_a
        for st in (s0, s1, s2, s3):
            st[5, sl] = zero
            st[6, sl] = zero
            st[7, sl] = zero
        sfc[R_LAB, sl] = lab
        sfc[R_LAC, sl] = lac
        sfc[R_LBC, sl] = lbc
        sfc[R_RAB, sl] = rab
        sfc[R_RAC, sl] = rac
        sfc[R_RBC, sl] = rbc
        sfc[R_RD, sl] = rd
        sfc[R_EBC, sl] = lab - m
        sfc[R_DG, sl] = dgate
        sfc[R_D, sl] = dd
        sfc[R_M, sl] = m
        for r in range(11, FCR):
            sfc[r, sl] = zero

    grp = base // F_BLK
    off = base % F_BLK
    col = grp * (4 * F_BLK) + off
    pltpu.sync_copy(s0, rhs.at[:, pl.ds(col, FPW)])
    pltpu.sync_copy(s1, rhs.at[:, pl.ds(col + F_BLK, FPW)])
    pltpu.sync_copy(s2, rhs.at[:, pl.ds(col + 2 * F_BLK, FPW)])
    pltpu.sync_copy(s3, rhs.at[:, pl.ds(col + 3 * F_BLK, FPW)])
    pltpu.sync_copy(sfc, fcs.at[:, pl.ds(base, FPW)])


@jax.jit
def _face_setup(vxh, vyh, vzh, fah, fbh, fch):
    kfn = pl.kernel(
        _sc_face_setup,
        out_type=(
            jax.ShapeDtypeStruct((8, 4 * F), jnp.float32),
            jax.ShapeDtypeStruct((FCR, F), jnp.float32),
        ),
        mesh=plsc.VectorSubcoreMesh(core_axis_name="c", subcore_axis_name="s"),
        scratch_types=[
            pltpu.VMEM((V,), jnp.float32),
            pltpu.VMEM((V,), jnp.float32),
            pltpu.VMEM((V,), jnp.float32),
            pltpu.VMEM((FPW,), jnp.int32),
            pltpu.VMEM((FPW,), jnp.int32),
            pltpu.VMEM((FPW,), jnp.int32),
            pltpu.VMEM((8, FPW), jnp.float32),
            pltpu.VMEM((8, FPW), jnp.float32),
            pltpu.VMEM((8, FPW), jnp.float32),
            pltpu.VMEM((8, FPW), jnp.float32),
            pltpu.VMEM((FCR, FPW), jnp.float32),
        ],
        compiler_params=pltpu.CompilerParams(needs_layout_passes=False),
    )
    return kfn(vxh, vyh, vzh, fah, fbh, fch)


def _tc_dist(pts_ref, rhs_ref, fc_ref, loss_ref, acc_ref):
    i = pl.program_id(0)
    j = pl.program_id(1)

    pts = pts_ref[...]
    col = jax.lax.broadcasted_iota(jnp.int32, (P_BLK, 8), 1)
    sq = jnp.where(col < 3, pts * pts, 0.0)
    pp = jnp.sum(sq, axis=1, keepdims=True)
    pts_aug = jnp.where(col == 3, pp, pts)

    g = jax.lax.dot_general(
        pts_aug, rhs_ref[...],
        dimension_numbers=(((1,), (0,)), ((), ())),
        preferred_element_type=jnp.float32,
        precision=jax.lax.Precision.HIGHEST,
    )
    d1 = g[:, 0:F_BLK]
    d2 = g[:, F_BLK:2 * F_BLK]
    ap2 = g[:, 2 * F_BLK:3 * F_BLK]
    hh = g[:, 3 * F_BLK:4 * F_BLK]

    row = lambda r: fc_ref[r:r + 1, :]
    lab = row(R_LAB)
    lac = row(R_LAC)
    lbc = row(R_LBC)
    m = row(R_M)

    vb = lac * d1 - m * d2
    vc = lab * d2 - m * d1
    va = row(R_D) - vb - vc
    inside = jnp.minimum(va, jnp.minimum(vb, vc)) >= 0.0
    df = (hh * hh) * row(R_RD) + row(R_DG)
    df = jnp.where(inside, df, 1e30)

    d1t = d1 + d1
    ta = jnp.clip(d1 * row(R_RAB), 0.0, 1.0)
    dab = ap2 - ta * (d1t - ta * lab)
    tc = jnp.clip(d2 * row(R_RAC), 0.0, 1.0)
    dac = ap2 - tc * (d2 + d2 - tc * lac)
    e = (d2 - d1) + row(R_EBC)
    bp2 = ap2 - d1t + lab
    tb = jnp.clip(e * row(R_RBC), 0.0, 1.0)
    dbc = bp2 - tb * (e + e - tb * lbc)

    d = jnp.minimum(jnp.minimum(dab, dac), jnp.minimum(dbc, df))
    dmin = jnp.min(d, axis=1, keepdims=True)

    @pl.when((i == 0) & (j == 0))
    def _():
        loss_ref[0, 0] = 0.0

    @pl.when(j == 0)
    def _():
        acc_ref[...] = dmin

    @pl.when(j > 0)
    def _():
        acc_ref[...] = jnp.minimum(acc_ref[...], dmin)

    @pl.when(j == NF - 1)
    def _():
        loss_ref[0, 0] += jnp.sum(acc_ref[...]) * (1.0 / N)


@functools.partial(jax.jit, static_argnames=("interpret",))
def _point_mesh_loss(pts8, rhs, fcs, interpret=False):
    out = pl.pallas_call(
        _tc_dist,
        grid=(NP, NF),
        in_specs=[
            pl.BlockSpec((P_BLK, 8), lambda i, j: (i, 0)),
            pl.BlockSpec((8, 4 * F_BLK), lambda i, j: (0, j)),
            pl.BlockSpec((FCR, F_BLK), lambda i, j: (0, j)),
        ],
        out_specs=pl.BlockSpec(memory_space=pltpu.SMEM),
        out_shape=jax.ShapeDtypeStruct((1, 1), jnp.float32),
        scratch_shapes=[pltpu.VMEM((P_BLK, 1), jnp.float32)],
        compiler_params=pltpu.CompilerParams(
            dimension_semantics=("arbitrary", "arbitrary"),
        ),
        interpret=interpret,
    )(pts8, rhs, fcs)
    return out[0, 0]


def kernel(body_verts, verts, faces):
    fi = faces.astype(jnp.int32)
    rhs, fcs = _face_setup(verts[:, 0], verts[:, 1], verts[:, 2],
                           fi[:, 0], fi[:, 1], fi[:, 2])
    pts = body_verts.reshape(-1, 3)
    pad = jnp.tile(jnp.array([[0.0, 1.0, 0.0, 0.0, 0.0]], jnp.float32), (N, 1))
    pts8 = jnp.concatenate([pts, pad], axis=1)
    return _point_mesh_loss(pts8, rhs, fcs)


# speedup vs baseline: 5.5755x; 1.1088x over previous
import functools

import jax
import jax.numpy as jnp
from jax.experimental import pallas as pl
from jax.experimental.pallas import tpu as pltpu
from jax.experimental.pallas import tpu_sc as plsc

F = 4096
V = 8192
N = 16384
NC, NS = 2, 16
NW = NC * NS
FPW = F // NW
FCR = 16

P_BLK = 512
F_BLK = 1024
NP = N // P_BLK
NF = F // F_BLK

R_LAB, R_LAC, R_LBC, R_RAB, R_RAC, R_RBC, R_RD, R_EBC, R_DG, R_D, R_M = range(11)


def _sc_face_setup(vxh, vyh, vzh, fah, fbh, fch, rhs, fcs,
                   vx, vy, vz, fa, fb, fc, s0, s1, s2, sfc):
    wid = jax.lax.axis_index("s") * NC + jax.lax.axis_index("c")
    base = wid * FPW
    pltpu.sync_copy(vxh, vx)
    pltpu.sync_copy(vyh, vy)
    pltpu.sync_copy(vzh, vz)
    pltpu.sync_copy(fah.at[pl.ds(base, FPW)], fa)
    pltpu.sync_copy(fbh.at[pl.ds(base, FPW)], fb)
    pltpu.sync_copy(fch.at[pl.ds(base, FPW)], fc)

    zero = jnp.zeros((16,), jnp.float32)
    one = jnp.ones((16,), jnp.float32)
    for j in range(FPW // 16):
        sl = pl.ds(j * 16, 16)
        ia = fa[sl]
        ib = fb[sl]
        ic = fc[sl]
        ax = plsc.load_gather(vx, [ia])
        ay = plsc.load_gather(vy, [ia])
        az = plsc.load_gather(vz, [ia])
        bx = plsc.load_gather(vx, [ib])
        by = plsc.load_gather(vy, [ib])
        bz = plsc.load_gather(vz, [ib])
        cx = plsc.load_gather(vx, [ic])
        cy = plsc.load_gather(vy, [ic])
        cz = plsc.load_gather(vz, [ic])
        abx, aby, abz = bx - ax, by - ay, bz - az
        acx, acy, acz = cx - ax, cy - ay, cz - az
        cbx, cby, cbz = cx - bx, cy - by, cz - bz
        nx = aby * acz - abz * acy
        ny = abz * acx - abx * acz
        nz = abx * acy - aby * acx
        kab_a = abx * ax + aby * ay + abz * az
        kac_a = acx * ax + acy * ay + acz * az
        na2 = ax * ax + ay * ay + az * az
        lab = abx * abx + aby * aby + abz * abz
        lac = acx * acx + acy * acy + acz * acz
        lbc = cbx * cbx + cby * cby + cbz * cbz
        m = abx * acx + aby * acy + abz * acz
        dd = nx * nx + ny * ny + nz * nz
        rab = one / jnp.where(lab == 0.0, one, lab)
        rac = one / jnp.where(lac == 0.0, one, lac)
        rbc = one / jnp.where(lbc == 0.0, one, lbc)
        rd = one / jnp.where(dd == 0.0, one, dd)
        dgate = jnp.where(dd > 1e-6 * (lab * lac), zero, jnp.full((16,), 1e30, jnp.float32))

        s0[0, sl] = abx
        s0[1, sl] = aby
        s0[2, sl] = abz
        s0[3, sl] = zero
        s0[4, sl] = -kab_a
        s1[0, sl] = acx
        s1[1, sl] = acy
        s1[2, sl] = acz
        s1[3, sl] = zero
        s1[4, sl] = -kac_a
        s2[0, sl] = -2.0 * ax
        s2[1, sl] = -2.0 * ay
        s2[2, sl] = -2.0 * az
        s2[3, sl] = one
        s2[4, sl] = na2
        for st in (s0, s1, s2):
            st[5, sl] = zero
            st[6, sl] = zero
            st[7, sl] = zero
        sfc[R_LAB, sl] = lab
        sfc[R_LAC, sl] = lac
        sfc[R_LBC, sl] = lbc
        sfc[R_RAB, sl] = rab
        sfc[R_RAC, sl] = rac
        sfc[R_RBC, sl] = rbc
        sfc[R_RD, sl] = rd
        sfc[R_EBC, sl] = lab - m
        sfc[R_DG, sl] = dgate
        sfc[R_D, sl] = dd
        sfc[R_M, sl] = m
        for r in range(11, FCR):
            sfc[r, sl] = zero

    grp = base // F_BLK
    off = base % F_BLK
    col = grp * (3 * F_BLK) + off
    pltpu.sync_copy(s0, rhs.at[:, pl.ds(col, FPW)])
    pltpu.sync_copy(s1, rhs.at[:, pl.ds(col + F_BLK, FPW)])
    pltpu.sync_copy(s2, rhs.at[:, pl.ds(col + 2 * F_BLK, FPW)])
    pltpu.sync_copy(sfc, fcs.at[:, pl.ds(base, FPW)])


@jax.jit
def _face_setup(vxh, vyh, vzh, fah, fbh, fch):
    kfn = pl.kernel(
        _sc_face_setup,
        out_type=(
            jax.ShapeDtypeStruct((8, 3 * F), jnp.float32),
            jax.ShapeDtypeStruct((FCR, F), jnp.float32),
        ),
        mesh=plsc.VectorSubcoreMesh(core_axis_name="c", subcore_axis_name="s"),
        scratch_types=[
            pltpu.VMEM((V,), jnp.float32),
            pltpu.VMEM((V,), jnp.float32),
            pltpu.VMEM((V,), jnp.float32),
            pltpu.VMEM((FPW,), jnp.int32),
            pltpu.VMEM((FPW,), jnp.int32),
            pltpu.VMEM((FPW,), jnp.int32),
            pltpu.VMEM((8, FPW), jnp.float32),
            pltpu.VMEM((8, FPW), jnp.float32),
            pltpu.VMEM((8, FPW), jnp.float32),
            pltpu.VMEM((FCR, FPW), jnp.float32),
        ],
        compiler_params=pltpu.CompilerParams(needs_layout_passes=False),
    )
    return kfn(vxh, vyh, vzh, fah, fbh, fch)


def _tc_dist(pts_ref, rhs_ref, fc_ref, loss_ref, acc_ref):
    i = pl.program_id(0)
    j = pl.program_id(1)

    pts = pts_ref[...]
    col = jax.lax.broadcasted_iota(jnp.int32, (P_BLK, 8), 1)
    sq = jnp.where(col < 3, pts * pts, 0.0)
    pp = jnp.sum(sq, axis=1, keepdims=True)
    pts_aug = jnp.where(col == 3, pp, pts)

    g = jax.lax.dot_general(
        pts_aug, rhs_ref[...],
        dimension_numbers=(((1,), (0,)), ((), ())),
        preferred_element_type=jnp.float32,
        precision=jax.lax.Precision.HIGHEST,
    )
    d1 = g[:, 0:F_BLK]
    d2 = g[:, F_BLK:2 * F_BLK]
    ap2 = g[:, 2 * F_BLK:3 * F_BLK]

    row = lambda r: fc_ref[r:r + 1, :]
    lab = row(R_LAB)
    lac = row(R_LAC)
    lbc = row(R_LBC)
    m = row(R_M)

    vb = lac * d1 - m * d2
    vc = lab * d2 - m * d1
    va = row(R_D) - vb - vc
    inside = jnp.minimum(va, jnp.minimum(vb, vc)) >= 0.0
    rd = row(R_RD)
    df = ap2 - ((vb * rd) * d1 + (vc * rd) * d2) + row(R_DG)
    df = jnp.where(inside, df, 1e30)

    d1t = d1 + d1
    ta = jnp.clip(d1 * row(R_RAB), 0.0, 1.0)
    dab = ap2 - ta * (d1t - ta * lab)
    tc = jnp.clip(d2 * row(R_RAC), 0.0, 1.0)
    dac = ap2 - tc * (d2 + d2 - tc * lac)
    e = (d2 - d1) + row(R_EBC)
    bp2 = ap2 - d1t + lab
    tb = jnp.clip(e * row(R_RBC), 0.0, 1.0)
    dbc = bp2 - tb * (e + e - tb * lbc)

    d = jnp.minimum(jnp.minimum(dab, dac), jnp.minimum(dbc, df))
    dmin = jnp.maximum(jnp.min(d, axis=1, keepdims=True), 0.0)

    @pl.when((i == 0) & (j == 0))
    def _():
        loss_ref[0, 0] = 0.0

    @pl.when(j == 0)
    def _():
        acc_ref[...] = dmin

    @pl.when(j > 0)
    def _():
        acc_ref[...] = jnp.minimum(acc_ref[...], dmin)

    @pl.when(j == NF - 1)
    def _():
        loss_ref[0, 0] += jnp.sum(acc_ref[...]) * (1.0 / N)


@functools.partial(jax.jit, static_argnames=("interpret",))
def _point_mesh_loss(pts8, rhs, fcs, interpret=False):
    out = pl.pallas_call(
        _tc_dist,
        grid=(NP, NF),
        in_specs=[
            pl.BlockSpec((P_BLK, 8), lambda i, j: (i, 0)),
            pl.BlockSpec((8, 3 * F_BLK), lambda i, j: (0, j)),
            pl.BlockSpec((FCR, F_BLK), lambda i, j: (0, j)),
        ],
        out_specs=pl.BlockSpec(memory_space=pltpu.SMEM),
        out_shape=jax.ShapeDtypeStruct((1, 1), jnp.float32),
        scratch_shapes=[pltpu.VMEM((P_BLK, 1), jnp.float32)],
        compiler_params=pltpu.CompilerParams(
            dimension_semantics=("arbitrary", "arbitrary"),
        ),
        interpret=interpret,
    )(pts8, rhs, fcs)
    return out[0, 0]


def kernel(body_verts, verts, faces):
    fi = faces.astype(jnp.int32)
    rhs, fcs = _face_setup(verts[:, 0], verts[:, 1], verts[:, 2],
                           fi[:, 0], fi[:, 1], fi[:, 2])
    pts = body_verts.reshape(-1, 3)
    pad = jnp.tile(jnp.array([[0.0, 1.0, 0.0, 0.0, 0.0]], jnp.float32), (N, 1))
    pts8 = jnp.concatenate([pts, pad], axis=1)
    return _point_mesh_loss(pts8, rhs, fcs)


# speedup vs baseline: 6.3970x; 1.1474x over previous
import functools

import jax
import jax.numpy as jnp
from jax.experimental import pallas as pl
from jax.experimental.pallas import tpu as pltpu
from jax.experimental.pallas import tpu_sc as plsc

F = 4096
V = 8192
N = 16384
NC, NS = 2, 16
NW = NC * NS
FPW = F // NW
FCR = 16

P_BLK = 512
F_BLK = 1024
NP = N // P_BLK
NF = F // F_BLK

R_LAB, R_LAC, R_LBC, R_RAB, R_RAC, R_RBC, R_RD, R_EBC, R_DG, R_D, R_M = range(11)


def _sc_face_setup(vxh, vyh, vzh, fah, fbh, fch, rhs, fcs,
                   vx, vy, vz, fa, fb, fc, s0, s1, s2, sfc):
    wid = jax.lax.axis_index("s") * NC + jax.lax.axis_index("c")
    base = wid * FPW
    pltpu.sync_copy(vxh, vx)
    pltpu.sync_copy(vyh, vy)
    pltpu.sync_copy(vzh, vz)
    pltpu.sync_copy(fah.at[pl.ds(base, FPW)], fa)
    pltpu.sync_copy(fbh.at[pl.ds(base, FPW)], fb)
    pltpu.sync_copy(fch.at[pl.ds(base, FPW)], fc)

    zero = jnp.zeros((16,), jnp.float32)
    one = jnp.ones((16,), jnp.float32)
    for j in range(FPW // 16):
        sl = pl.ds(j * 16, 16)
        ia = fa[sl]
        ib = fb[sl]
        ic = fc[sl]
        ax = plsc.load_gather(vx, [ia])
        ay = plsc.load_gather(vy, [ia])
        az = plsc.load_gather(vz, [ia])
        bx = plsc.load_gather(vx, [ib])
        by = plsc.load_gather(vy, [ib])
        bz = plsc.load_gather(vz, [ib])
        cx = plsc.load_gather(vx, [ic])
        cy = plsc.load_gather(vy, [ic])
        cz = plsc.load_gather(vz, [ic])
        abx, aby, abz = bx - ax, by - ay, bz - az
        acx, acy, acz = cx - ax, cy - ay, cz - az
        cbx, cby, cbz = cx - bx, cy - by, cz - bz
        nx = aby * acz - abz * acy
        ny = abz * acx - abx * acz
        nz = abx * acy - aby * acx
        kab_a = abx * ax + aby * ay + abz * az
        kac_a = acx * ax + acy * ay + acz * az
        na2 = ax * ax + ay * ay + az * az
        lab = abx * abx + aby * aby + abz * abz
        lac = acx * acx + acy * acy + acz * acz
        lbc = cbx * cbx + cby * cby + cbz * cbz
        m = abx * acx + aby * acy + abz * acz
        dd = nx * nx + ny * ny + nz * nz
        rab = one / jnp.where(lab == 0.0, one, lab)
        rac = one / jnp.where(lac == 0.0, one, lac)
        rbc = one / jnp.where(lbc == 0.0, one, lbc)
        rd = one / jnp.where(dd == 0.0, one, dd)
        dgate = jnp.where(dd > 1e-6 * (lab * lac), zero, jnp.full((16,), 1e30, jnp.float32))

        s0[0, sl] = abx
        s0[1, sl] = aby
        s0[2, sl] = abz
        s0[3, sl] = zero
        s0[4, sl] = -kab_a
        s1[0, sl] = acx
        s1[1, sl] = acy
        s1[2, sl] = acz
        s1[3, sl] = zero
        s1[4, sl] = -kac_a
        s2[0, sl] = -2.0 * ax
        s2[1, sl] = -2.0 * ay
        s2[2, sl] = -2.0 * az
        s2[3, sl] = one
        s2[4, sl] = na2
        for st in (s0, s1, s2):
            st[5, sl] = zero
            st[6, sl] = zero
            st[7, sl] = zero
        sfc[R_LAB, sl] = lab
        sfc[R_LAC, sl] = lac
        sfc[R_LBC, sl] = lbc
        sfc[R_RAB, sl] = rab
        sfc[R_RAC, sl] = rac
        sfc[R_RBC, sl] = rbc
        sfc[R_RD, sl] = rd
        sfc[R_EBC, sl] = lab - m
        sfc[R_DG, sl] = dgate
        sfc[R_D, sl] = dd
        sfc[R_M, sl] = m
        for r in range(11, FCR):
            sfc[r, sl] = zero

    grp = base // F_BLK
    off = base % F_BLK
    col = grp * (3 * F_BLK) + off
    pltpu.sync_copy(s0, rhs.at[:, pl.ds(col, FPW)])
    pltpu.sync_copy(s1, rhs.at[:, pl.ds(col + F_BLK, FPW)])
    pltpu.sync_copy(s2, rhs.at[:, pl.ds(col + 2 * F_BLK, FPW)])
    pltpu.sync_copy(sfc, fcs.at[:, pl.ds(base, FPW)])


@jax.jit
def _face_setup(vxh, vyh, vzh, fah, fbh, fch):
    kfn = pl.kernel(
        _sc_face_setup,
        out_type=(
            jax.ShapeDtypeStruct((8, 3 * F), jnp.float32),
            jax.ShapeDtypeStruct((FCR, F), jnp.float32),
        ),
        mesh=plsc.VectorSubcoreMesh(core_axis_name="c", subcore_axis_name="s"),
        scratch_types=[
            pltpu.VMEM((V,), jnp.float32),
            pltpu.VMEM((V,), jnp.float32),
            pltpu.VMEM((V,), jnp.float32),
            pltpu.VMEM((FPW,), jnp.int32),
            pltpu.VMEM((FPW,), jnp.int32),
            pltpu.VMEM((FPW,), jnp.int32),
            pltpu.VMEM((8, FPW), jnp.float32),
            pltpu.VMEM((8, FPW), jnp.float32),
            pltpu.VMEM((8, FPW), jnp.float32),
            pltpu.VMEM((FCR, FPW), jnp.float32),
        ],
        compiler_params=pltpu.CompilerParams(needs_layout_passes=False),
    )
    return kfn(vxh, vyh, vzh, fah, fbh, fch)


def _tc_dist(pts_ref, rhs_ref, fc_ref, loss_ref, acc_ref):
    i = pl.program_id(0)
    j = pl.program_id(1)

    pts = pts_ref[...]
    col = jax.lax.broadcasted_iota(jnp.int32, (P_BLK, 8), 1)
    sq = jnp.where(col < 3, pts * pts, 0.0)
    pp = jnp.sum(sq, axis=1, keepdims=True)
    pts_aug = jnp.where(col == 3, pp, pts)

    rhsf = rhs_ref[...]
    ah = pts_aug.astype(jnp.bfloat16)
    al = (pts_aug - ah.astype(jnp.float32)).astype(jnp.bfloat16)
    bh = rhsf.astype(jnp.bfloat16)
    bl = (rhsf - bh.astype(jnp.float32)).astype(jnp.bfloat16)
    dd = lambda x, y: jax.lax.dot_general(
        x, y, (((1,), (0,)), ((), ())), preferred_element_type=jnp.float32)
    g = dd(ah, bh) + dd(ah, bl) + dd(al, bh)
    d1 = g[:, 0:F_BLK]
    d2 = g[:, F_BLK:2 * F_BLK]
    ap2 = g[:, 2 * F_BLK:3 * F_BLK]

    row = lambda r: fc_ref[r:r + 1, :]
    lab = row(R_LAB)
    lac = row(R_LAC)
    lbc = row(R_LBC)
    m = row(R_M)

    vb = lac * d1 - m * d2
    vc = lab * d2 - m * d1
    va = row(R_D) - vb - vc
    inside = jnp.minimum(va, jnp.minimum(vb, vc)) >= 0.0
    rd = row(R_RD)
    df = ap2 - ((vb * rd) * d1 + (vc * rd) * d2) + row(R_DG)
    df = jnp.where(inside, df, 1e30)

    d1t = d1 + d1
    ta = jnp.clip(d1 * row(R_RAB), 0.0, 1.0)
    dab = ap2 - ta * (d1t - ta * lab)
    tc = jnp.clip(d2 * row(R_RAC), 0.0, 1.0)
    dac = ap2 - tc * (d2 + d2 - tc * lac)
    e = (d2 - d1) + row(R_EBC)
    bp2 = ap2 - d1t + lab
    tb = jnp.clip(e * row(R_RBC), 0.0, 1.0)
    dbc = bp2 - tb * (e + e - tb * lbc)

    d = jnp.minimum(jnp.minimum(dab, dac), jnp.minimum(dbc, df))
    dmin = jnp.maximum(jnp.min(d, axis=1, keepdims=True), 0.0)

    @pl.when((i == 0) & (j == 0))
    def _():
        loss_ref[0, 0] = 0.0

    @pl.when(j == 0)
    def _():
        acc_ref[...] = dmin

    @pl.when(j > 0)
    def _():
        acc_ref[...] = jnp.minimum(acc_ref[...], dmin)

    @pl.when(j == NF - 1)
    def _():
        loss_ref[0, 0] += jnp.sum(acc_ref[...]) * (1.0 / N)


@functools.partial(jax.jit, static_argnames=("interpret",))
def _point_mesh_loss(pts8, rhs, fcs, interpret=False):
    out = pl.pallas_call(
        _tc_dist,
        grid=(NP, NF),
        in_specs=[
            pl.BlockSpec((P_BLK, 8), lambda i, j: (i, 0)),
            pl.BlockSpec((8, 3 * F_BLK), lambda i, j: (0, j)),
            pl.BlockSpec((FCR, F_BLK), lambda i, j: (0, j)),
        ],
        out_specs=pl.BlockSpec(memory_space=pltpu.SMEM),
        out_shape=jax.ShapeDtypeStruct((1, 1), jnp.float32),
        scratch_shapes=[pltpu.VMEM((P_BLK, 1), jnp.float32)],
        compiler_params=pltpu.CompilerParams(
            dimension_semantics=("arbitrary", "arbitrary"),
        ),
        interpret=interpret,
    )(pts8, rhs, fcs)
    return out[0, 0]


def kernel(body_verts, verts, faces):
    fi = faces.astype(jnp.int32)
    rhs, fcs = _face_setup(verts[:, 0], verts[:, 1], verts[:, 2],
                           fi[:, 0], fi[:, 1], fi[:, 2])
    pts = body_verts.reshape(-1, 3)
    pad = jnp.tile(jnp.array([[0.0, 1.0, 0.0, 0.0, 0.0]], jnp.float32), (N, 1))
    pts8 = jnp.concatenate([pts, pad], axis=1)
    return _point_mesh_loss(pts8, rhs, fcs)


# speedup vs baseline: 6.4376x; 1.0063x over previous
import functools

import jax
import jax.numpy as jnp
from jax.experimental import pallas as pl
from jax.experimental.pallas import tpu as pltpu
from jax.experimental.pallas import tpu_sc as plsc

F = 4096
V = 8192
N = 16384
NC, NS = 2, 16
NW = NC * NS
FPW = F // NW
FCR = 16

P_BLK = 512
F_BLK = 2048
NP = N // P_BLK
NF = F // F_BLK

R_LAB, R_LAC, R_LBC, R_RAB, R_RAC, R_RBC, R_RD, R_EBC, R_DG, R_D, R_M = range(11)


def _sc_face_setup(vxh, vyh, vzh, fah, fbh, fch, rhs, fcs,
                   vx, vy, vz, fa, fb, fc, s0, s1, s2, sfc):
    wid = jax.lax.axis_index("s") * NC + jax.lax.axis_index("c")
    base = wid * FPW
    pltpu.sync_copy(vxh, vx)
    pltpu.sync_copy(vyh, vy)
    pltpu.sync_copy(vzh, vz)
    pltpu.sync_copy(fah.at[pl.ds(base, FPW)], fa)
    pltpu.sync_copy(fbh.at[pl.ds(base, FPW)], fb)
    pltpu.sync_copy(fch.at[pl.ds(base, FPW)], fc)

    zero = jnp.zeros((16,), jnp.float32)
    one = jnp.ones((16,), jnp.float32)
    for j in range(FPW // 16):
        sl = pl.ds(j * 16, 16)
        ia = fa[sl]
        ib = fb[sl]
        ic = fc[sl]
        ax = plsc.load_gather(vx, [ia])
        ay = plsc.load_gather(vy, [ia])
        az = plsc.load_gather(vz, [ia])
        bx = plsc.load_gather(vx, [ib])
        by = plsc.load_gather(vy, [ib])
        bz = plsc.load_gather(vz, [ib])
        cx = plsc.load_gather(vx, [ic])
        cy = plsc.load_gather(vy, [ic])
        cz = plsc.load_gather(vz, [ic])
        abx, aby, abz = bx - ax, by - ay, bz - az
        acx, acy, acz = cx - ax, cy - ay, cz - az
        cbx, cby, cbz = cx - bx, cy - by, cz - bz
        nx = aby * acz - abz * acy
        ny = abz * acx - abx * acz
        nz = abx * acy - aby * acx
        kab_a = abx * ax + aby * ay + abz * az
        kac_a = acx * ax + acy * ay + acz * az
        na2 = ax * ax + ay * ay + az * az
        lab = abx * abx + aby * aby + abz * abz
        lac = acx * acx + acy * acy + acz * acz
        lbc = cbx * cbx + cby * cby + cbz * cbz
        m = abx * acx + aby * acy + abz * acz
        dd = nx * nx + ny * ny + nz * nz
        rab = one / jnp.where(lab == 0.0, one, lab)
        rac = one / jnp.where(lac == 0.0, one, lac)
        rbc = one / jnp.where(lbc == 0.0, one, lbc)
        rd = one / jnp.where(dd == 0.0, one, dd)
        dgate = jnp.where(dd > 1e-6 * (lab * lac), zero, jnp.full((16,), 1e30, jnp.float32))

        s0[0, sl] = abx
        s0[1, sl] = aby
        s0[2, sl] = abz
        s0[3, sl] = zero
        s0[4, sl] = -kab_a
        s1[0, sl] = acx
        s1[1, sl] = acy
        s1[2, sl] = acz
        s1[3, sl] = zero
        s1[4, sl] = -kac_a
        s2[0, sl] = -2.0 * ax
        s2[1, sl] = -2.0 * ay
        s2[2, sl] = -2.0 * az
        s2[3, sl] = one
        s2[4, sl] = na2
        for st in (s0, s1, s2):
            st[5, sl] = zero
            st[6, sl] = zero
            st[7, sl] = zero
        sfc[R_LAB, sl] = lab
        sfc[R_LAC, sl] = lac
        sfc[R_LBC, sl] = lbc
        sfc[R_RAB, sl] = rab
        sfc[R_RAC, sl] = rac
        sfc[R_RBC, sl] = rbc
        sfc[R_RD, sl] = rd
        sfc[R_EBC, sl] = lab - m
        sfc[R_DG, sl] = dgate
        sfc[R_D, sl] = dd
        sfc[R_M, sl] = m
        for r in range(11, FCR):
            sfc[r, sl] = zero

    grp = base // F_BLK
    off = base % F_BLK
    col = grp * (3 * F_BLK) + off
    pltpu.sync_copy(s0, rhs.at[:, pl.ds(col, FPW)])
    pltpu.sync_copy(s1, rhs.at[:, pl.ds(col + F_BLK, FPW)])
    pltpu.sync_copy(s2, rhs.at[:, pl.ds(col + 2 * F_BLK, FPW)])
    pltpu.sync_copy(sfc, fcs.at[:, pl.ds(base, FPW)])


@jax.jit
def _face_setup(vxh, vyh, vzh, fah, fbh, fch):
    kfn = pl.kernel(
        _sc_face_setup,
        out_type=(
            jax.ShapeDtypeStruct((8, 3 * F), jnp.float32),
            jax.ShapeDtypeStruct((FCR, F), jnp.float32),
        ),
        mesh=plsc.VectorSubcoreMesh(core_axis_name="c", subcore_axis_name="s"),
        scratch_types=[
            pltpu.VMEM((V,), jnp.float32),
            pltpu.VMEM((V,), jnp.float32),
            pltpu.VMEM((V,), jnp.float32),
            pltpu.VMEM((FPW,), jnp.int32),
            pltpu.VMEM((FPW,), jnp.int32),
            pltpu.VMEM((FPW,), jnp.int32),
            pltpu.VMEM((8, FPW), jnp.float32),
            pltpu.VMEM((8, FPW), jnp.float32),
            pltpu.VMEM((8, FPW), jnp.float32),
            pltpu.VMEM((FCR, FPW), jnp.float32),
        ],
        compiler_params=pltpu.CompilerParams(needs_layout_passes=False),
    )
    return kfn(vxh, vyh, vzh, fah, fbh, fch)


def _tc_dist(pts_ref, rhs_ref, fc_ref, loss_ref, acc_ref):
    i = pl.program_id(0)
    j = pl.program_id(1)

    pts = pts_ref[...]
    col = jax.lax.broadcasted_iota(jnp.int32, (P_BLK, 8), 1)
    sq = jnp.where(col < 3, pts * pts, 0.0)
    pp = jnp.sum(sq, axis=1, keepdims=True)
    pts_aug = jnp.where(col == 3, pp, pts)

    rhsf = rhs_ref[...]
    ah = pts_aug.astype(jnp.bfloat16)
    al = (pts_aug - ah.astype(jnp.float32)).astype(jnp.bfloat16)
    bh = rhsf.astype(jnp.bfloat16)
    bl = (rhsf - bh.astype(jnp.float32)).astype(jnp.bfloat16)
    dd = lambda x, y: jax.lax.dot_general(
        x, y, (((1,), (0,)), ((), ())), preferred_element_type=jnp.float32)
    g = dd(ah, bh) + dd(ah, bl) + dd(al, bh)
    d1 = g[:, 0:F_BLK]
    d2 = g[:, F_BLK:2 * F_BLK]
    ap2 = g[:, 2 * F_BLK:3 * F_BLK]

    row = lambda r: fc_ref[r:r + 1, :]
    lab = row(R_LAB)
    lac = row(R_LAC)
    lbc = row(R_LBC)
    m = row(R_M)

    vb = lac * d1 - m * d2
    vc = lab * d2 - m * d1
    va = row(R_D) - vb - vc
    inside = jnp.minimum(va, jnp.minimum(vb, vc)) >= 0.0
    rd = row(R_RD)
    df = ap2 - ((vb * rd) * d1 + (vc * rd) * d2) + row(R_DG)
    df = jnp.where(inside, df, 1e30)

    d1t = d1 + d1
    ta = jnp.clip(d1 * row(R_RAB), 0.0, 1.0)
    dab = ap2 - ta * (d1t - ta * lab)
    tc = jnp.clip(d2 * row(R_RAC), 0.0, 1.0)
    dac = ap2 - tc * (d2 + d2 - tc * lac)
    e = (d2 - d1) + row(R_EBC)
    bp2 = ap2 - d1t + lab
    tb = jnp.clip(e * row(R_RBC), 0.0, 1.0)
    dbc = bp2 - tb * (e + e - tb * lbc)

    d = jnp.minimum(jnp.minimum(dab, dac), jnp.minimum(dbc, df))
    dmin = jnp.maximum(jnp.min(d, axis=1, keepdims=True), 0.0)

    @pl.when((i == 0) & (j == 0))
    def _():
        loss_ref[0, 0] = 0.0

    @pl.when(j == 0)
    def _():
        acc_ref[...] = dmin

    @pl.when(j > 0)
    def _():
        acc_ref[...] = jnp.minimum(acc_ref[...], dmin)

    @pl.when(j == NF - 1)
    def _():
        loss_ref[0, 0] += jnp.sum(acc_ref[...]) * (1.0 / N)


@functools.partial(jax.jit, static_argnames=("interpret",))
def _point_mesh_loss(pts8, rhs, fcs, interpret=False):
    out = pl.pallas_call(
        _tc_dist,
        grid=(NP, NF),
        in_specs=[
            pl.BlockSpec((P_BLK, 8), lambda i, j: (i, 0)),
            pl.BlockSpec((8, 3 * F_BLK), lambda i, j: (0, j)),
            pl.BlockSpec((FCR, F_BLK), lambda i, j: (0, j)),
        ],
        out_specs=pl.BlockSpec(memory_space=pltpu.SMEM),
        out_shape=jax.ShapeDtypeStruct((1, 1), jnp.float32),
        scratch_shapes=[pltpu.VMEM((P_BLK, 1), jnp.float32)],
        compiler_params=pltpu.CompilerParams(
            dimension_semantics=("arbitrary", "arbitrary"),
        ),
        interpret=interpret,
    )(pts8, rhs, fcs)
    return out[0, 0]


def kernel(body_verts, verts, faces):
    fi = faces.astype(jnp.int32)
    rhs, fcs = _face_setup(verts[:, 0], verts[:, 1], verts[:, 2],
                           fi[:, 0], fi[:, 1], fi[:, 2])
    pts = body_verts.reshape(-1, 3)
    pad = jnp.tile(jnp.array([[0.0, 1.0, 0.0, 0.0, 0.0]], jnp.float32), (N, 1))
    pts8 = jnp.concatenate([pts, pad], axis=1)
    return _point_mesh_loss(pts8, rhs, fcs)


# speedup vs baseline: 6.4779x; 1.0063x over previous
import functools

import jax
import jax.numpy as jnp
from jax.experimental import pallas as pl
from jax.experimental.pallas import tpu as pltpu
from jax.experimental.pallas import tpu_sc as plsc

F = 4096
V = 8192
N = 16384
NC, NS = 2, 16
NW = NC * NS
FPW = F // NW
FCR = 16

P_BLK = 1024
F_BLK = 2048
NP = N // P_BLK
NF = F // F_BLK

R_LAB, R_LAC, R_LBC, R_RAB, R_RAC, R_RBC, R_RD, R_EBC, R_DG, R_D, R_M = range(11)


def _sc_face_setup(vxh, vyh, vzh, fah, fbh, fch, rhs, fcs,
                   vx, vy, vz, fa, fb, fc, s0, s1, s2, sfc):
    wid = jax.lax.axis_index("s") * NC + jax.lax.axis_index("c")
    base = wid * FPW
    pltpu.sync_copy(vxh, vx)
    pltpu.sync_copy(vyh, vy)
    pltpu.sync_copy(vzh, vz)
    pltpu.sync_copy(fah.at[pl.ds(base, FPW)], fa)
    pltpu.sync_copy(fbh.at[pl.ds(base, FPW)], fb)
    pltpu.sync_copy(fch.at[pl.ds(base, FPW)], fc)

    zero = jnp.zeros((16,), jnp.float32)
    one = jnp.ones((16,), jnp.float32)
    for j in range(FPW // 16):
        sl = pl.ds(j * 16, 16)
        ia = fa[sl]
        ib = fb[sl]
        ic = fc[sl]
        ax = plsc.load_gather(vx, [ia])
        ay = plsc.load_gather(vy, [ia])
        az = plsc.load_gather(vz, [ia])
        bx = plsc.load_gather(vx, [ib])
        by = plsc.load_gather(vy, [ib])
        bz = plsc.load_gather(vz, [ib])
        cx = plsc.load_gather(vx, [ic])
        cy = plsc.load_gather(vy, [ic])
        cz = plsc.load_gather(vz, [ic])
        abx, aby, abz = bx - ax, by - ay, bz - az
        acx, acy, acz = cx - ax, cy - ay, cz - az
        cbx, cby, cbz = cx - bx, cy - by, cz - bz
        nx = aby * acz - abz * acy
        ny = abz * acx - abx * acz
        nz = abx * acy - aby * acx
        kab_a = abx * ax + aby * ay + abz * az
        kac_a = acx * ax + acy * ay + acz * az
        na2 = ax * ax + ay * ay + az * az
        lab = abx * abx + aby * aby + abz * abz
        lac = acx * acx + acy * acy + acz * acz
        lbc = cbx * cbx + cby * cby + cbz * cbz
        m = abx * acx + aby * acy + abz * acz
        dd = nx * nx + ny * ny + nz * nz
        rab = one / jnp.where(lab == 0.0, one, lab)
        rac = one / jnp.where(lac == 0.0, one, lac)
        rbc = one / jnp.where(lbc == 0.0, one, lbc)
        rd = one / jnp.where(dd == 0.0, one, dd)
        dgate = jnp.where(dd > 1e-6 * (lab * lac), zero, jnp.full((16,), 1e30, jnp.float32))

        s0[0, sl] = abx
        s0[1, sl] = aby
        s0[2, sl] = abz
        s0[3, sl] = zero
        s0[4, sl] = -kab_a
        s1[0, sl] = acx
        s1[1, sl] = acy
        s1[2, sl] = acz
        s1[3, sl] = zero
        s1[4, sl] = -kac_a
        s2[0, sl] = -2.0 * ax
        s2[1, sl] = -2.0 * ay
        s2[2, sl] = -2.0 * az
        s2[3, sl] = one
        s2[4, sl] = na2
        for st in (s0, s1, s2):
            st[5, sl] = zero
            st[6, sl] = zero
            st[7, sl] = zero
        sfc[R_LAB, sl] = lab
        sfc[R_LAC, sl] = lac
        sfc[R_LBC, sl] = lbc
        sfc[R_RAB, sl] = rab
        sfc[R_RAC, sl] = rac
        sfc[R_RBC, sl] = rbc
        sfc[R_RD, sl] = rd
        sfc[R_EBC, sl] = lab - m
        sfc[R_DG, sl] = dgate
        sfc[R_D, sl] = dd
        sfc[R_M, sl] = m
        for r in range(11, FCR):
            sfc[r, sl] = zero

    grp = base // F_BLK
    off = base % F_BLK
    col = grp * (3 * F_BLK) + off
    pltpu.sync_copy(s0, rhs.at[:, pl.ds(col, FPW)])
    pltpu.sync_copy(s1, rhs.at[:, pl.ds(col + F_BLK, FPW)])
    pltpu.sync_copy(s2, rhs.at[:, pl.ds(col + 2 * F_BLK, FPW)])
    pltpu.sync_copy(sfc, fcs.at[:, pl.ds(base, FPW)])


@jax.jit
def _face_setup(vxh, vyh, vzh, fah, fbh, fch):
    kfn = pl.kernel(
        _sc_face_setup,
        out_type=(
            jax.ShapeDtypeStruct((8, 3 * F), jnp.float32),
            jax.ShapeDtypeStruct((FCR, F), jnp.float32),
        ),
        mesh=plsc.VectorSubcoreMesh(core_axis_name="c", subcore_axis_name="s"),
        scratch_types=[
            pltpu.VMEM((V,), jnp.float32),
            pltpu.VMEM((V,), jnp.float32),
            pltpu.VMEM((V,), jnp.float32),
            pltpu.VMEM((FPW,), jnp.int32),
            pltpu.VMEM((FPW,), jnp.int32),
            pltpu.VMEM((FPW,), jnp.int32),
            pltpu.VMEM((8, FPW), jnp.float32),
            pltpu.VMEM((8, FPW), jnp.float32),
            pltpu.VMEM((8, FPW), jnp.float32),
            pltpu.VMEM((FCR, FPW), jnp.float32),
        ],
        compiler_params=pltpu.CompilerParams(needs_layout_passes=False),
    )
    return kfn(vxh, vyh, vzh, fah, fbh, fch)


def _tc_dist(pts_ref, rhs_ref, fc_ref, loss_ref, acc_ref):
    i = pl.program_id(0)
    j = pl.program_id(1)

    pts = pts_ref[...]
    col = jax.lax.broadcasted_iota(jnp.int32, (P_BLK, 8), 1)
    sq = jnp.where(col < 3, pts * pts, 0.0)
    pp = jnp.sum(sq, axis=1, keepdims=True)
    pts_aug = jnp.where(col == 3, pp, pts)

    rhsf = rhs_ref[...]
    ah = pts_aug.astype(jnp.bfloat16)
    al = (pts_aug - ah.astype(jnp.float32)).astype(jnp.bfloat16)
    bh = rhsf.astype(jnp.bfloat16)
    bl = (rhsf - bh.astype(jnp.float32)).astype(jnp.bfloat16)
    dd = lambda x, y: jax.lax.dot_general(
        x, y, (((1,), (0,)), ((), ())), preferred_element_type=jnp.float32)
    g = dd(ah, bh) + dd(ah, bl) + dd(al, bh)
    d1 = g[:, 0:F_BLK]
    d2 = g[:, F_BLK:2 * F_BLK]
    ap2 = g[:, 2 * F_BLK:3 * F_BLK]

    row = lambda r: fc_ref[r:r + 1, :]
    lab = row(R_LAB)
    lac = row(R_LAC)
    lbc = row(R_LBC)
    m = row(R_M)

    vb = lac * d1 - m * d2
    vc = lab * d2 - m * d1
    va = row(R_D) - vb - vc
    inside = jnp.minimum(va, jnp.minimum(vb, vc)) >= 0.0
    rd = row(R_RD)
    df = ap2 - ((vb * rd) * d1 + (vc * rd) * d2) + row(R_DG)
    df = jnp.where(inside, df, 1e30)

    d1t = d1 + d1
    ta = jnp.clip(d1 * row(R_RAB), 0.0, 1.0)
    dab = ap2 - ta * (d1t - ta * lab)
    tc = jnp.clip(d2 * row(R_RAC), 0.0, 1.0)
    dac = ap2 - tc * (d2 + d2 - tc * lac)
    e = (d2 - d1) + row(R_EBC)
    bp2 = ap2 - d1t + lab
    tb = jnp.clip(e * row(R_RBC), 0.0, 1.0)
    dbc = bp2 - tb * (e + e - tb * lbc)

    d = jnp.minimum(jnp.minimum(dab, dac), jnp.minimum(dbc, df))
    dmin = jnp.maximum(jnp.min(d, axis=1, keepdims=True), 0.0)

    @pl.when((i == 0) & (j == 0))
    def _():
        loss_ref[0, 0] = 0.0

    @pl.when(j == 0)
    def _():
        acc_ref[...] = dmin

    @pl.when(j > 0)
    def _():
        acc_ref[...] = jnp.minimum(acc_ref[...], dmin)

    @pl.when(j == NF - 1)
    def _():
        loss_ref[0, 0] += jnp.sum(acc_ref[...]) * (1.0 / N)


@functools.partial(jax.jit, static_argnames=("interpret",))
def _point_mesh_loss(pts8, rhs, fcs, interpret=False):
    out = pl.pallas_call(
        _tc_dist,
        grid=(NP, NF),
        in_specs=[
            pl.BlockSpec((P_BLK, 8), lambda i, j: (i, 0)),
            pl.BlockSpec((8, 3 * F_BLK), lambda i, j: (0, j)),
            pl.BlockSpec((FCR, F_BLK), lambda i, j: (0, j)),
        ],
        out_specs=pl.BlockSpec(memory_space=pltpu.SMEM),
        out_shape=jax.ShapeDtypeStruct((1, 1), jnp.float32),
        scratch_shapes=[pltpu.VMEM((P_BLK, 1), jnp.float32)],
        compiler_params=pltpu.CompilerParams(
            dimension_semantics=("arbitrary", "arbitrary"),
        ),
        interpret=interpret,
    )(pts8, rhs, fcs)
    return out[0, 0]


def kernel(body_verts, verts, faces):
    fi = faces.astype(jnp.int32)
    rhs, fcs = _face_setup(verts[:, 0], verts[:, 1], verts[:, 2],
                           fi[:, 0], fi[:, 1], fi[:, 2])
    pts = body_verts.reshape(-1, 3)
    pad = jnp.tile(jnp.array([[0.0, 1.0, 0.0, 0.0, 0.0]], jnp.float32), (N, 1))
    pts8 = jnp.concatenate([pts, pad], axis=1)
    return _point_mesh_loss(pts8, rhs, fcs)


# speedup vs baseline: 6.8526x; 1.0578x over previous
import functools

import jax
import jax.numpy as jnp
from jax.experimental import pallas as pl
from jax.experimental.pallas import tpu as pltpu
from jax.experimental.pallas import tpu_sc as plsc

F = 4096
V = 8192
N = 16384
NC, NS = 2, 16
NW = NC * NS
FPW = F // NW
FCR = 16

P_BLK = 512
F_BLK = 2048
NP = N // P_BLK
NF = F // F_BLK

R_LAB, R_LAC, R_LBC, R_RAB, R_RAC, R_RBC, R_RD, R_EBC, R_DG, R_D, R_M = range(11)


def _sc_face_setup(vxh, vyh, vzh, fah, fbh, fch, rhs, fcs,
                   vx, vy, vz, fa, fb, fc, s0, s1, s2, sfc):
    wid = jax.lax.axis_index("s") * NC + jax.lax.axis_index("c")
    base = wid * FPW
    pltpu.sync_copy(vxh, vx)
    pltpu.sync_copy(vyh, vy)
    pltpu.sync_copy(vzh, vz)
    pltpu.sync_copy(fah.at[pl.ds(base, FPW)], fa)
    pltpu.sync_copy(fbh.at[pl.ds(base, FPW)], fb)
    pltpu.sync_copy(fch.at[pl.ds(base, FPW)], fc)

    zero = jnp.zeros((16,), jnp.float32)
    one = jnp.ones((16,), jnp.float32)
    for j in range(FPW // 16):
        sl = pl.ds(j * 16, 16)
        ia = fa[sl]
        ib = fb[sl]
        ic = fc[sl]
        ax = plsc.load_gather(vx, [ia])
        ay = plsc.load_gather(vy, [ia])
        az = plsc.load_gather(vz, [ia])
        bx = plsc.load_gather(vx, [ib])
        by = plsc.load_gather(vy, [ib])
        bz = plsc.load_gather(vz, [ib])
        cx = plsc.load_gather(vx, [ic])
        cy = plsc.load_gather(vy, [ic])
        cz = plsc.load_gather(vz, [ic])
        abx, aby, abz = bx - ax, by - ay, bz - az
        acx, acy, acz = cx - ax, cy - ay, cz - az
        cbx, cby, cbz = cx - bx, cy - by, cz - bz
        nx = aby * acz - abz * acy
        ny = abz * acx - abx * acz
        nz = abx * acy - aby * acx
        kab_a = abx * ax + aby * ay + abz * az
        kac_a = acx * ax + acy * ay + acz * az
        na2 = ax * ax + ay * ay + az * az
        lab = abx * abx + aby * aby + abz * abz
        lac = acx * acx + acy * acy + acz * acz
        lbc = cbx * cbx + cby * cby + cbz * cbz
        m = abx * acx + aby * acy + abz * acz
        dd = nx * nx + ny * ny + nz * nz
        rab = one / jnp.where(lab == 0.0, one, lab)
        rac = one / jnp.where(lac == 0.0, one, lac)
        rbc = one / jnp.where(lbc == 0.0, one, lbc)
        rd = one / jnp.where(dd == 0.0, one, dd)
        dgate = jnp.where(dd > 1e-6 * (lab * lac), zero, jnp.full((16,), 1e30, jnp.float32))

        s0[0, sl] = abx
        s0[1, sl] = aby
        s0[2, sl] = abz
        s0[3, sl] = zero
        s0[4, sl] = -kab_a
        s1[0, sl] = acx
        s1[1, sl] = acy
        s1[2, sl] = acz
        s1[3, sl] = zero
        s1[4, sl] = -kac_a
        s2[0, sl] = -2.0 * ax
        s2[1, sl] = -2.0 * ay
        s2[2, sl] = -2.0 * az
        s2[3, sl] = one
        s2[4, sl] = na2
        for st in (s0, s1, s2):
            st[5, sl] = zero
            st[6, sl] = zero
            st[7, sl] = zero
        sfc[R_LAB, sl] = lab
        sfc[R_LAC, sl] = lac
        sfc[R_LBC, sl] = lbc
        sfc[R_RAB, sl] = rab
        sfc[R_RAC, sl] = rac
        sfc[R_RBC, sl] = rbc
        sfc[R_RD, sl] = rd
        sfc[R_EBC, sl] = lab - m
        sfc[R_DG, sl] = dgate
        sfc[R_D, sl] = dd
        sfc[R_M, sl] = m
        for r in range(11, FCR):
            sfc[r, sl] = zero

    grp = base // F_BLK
    off = base % F_BLK
    col = grp * (3 * F_BLK) + off
    pltpu.sync_copy(s0, rhs.at[:, pl.ds(col, FPW)])
    pltpu.sync_copy(s1, rhs.at[:, pl.ds(col + F_BLK, FPW)])
    pltpu.sync_copy(s2, rhs.at[:, pl.ds(col + 2 * F_BLK, FPW)])
    pltpu.sync_copy(sfc, fcs.at[:, pl.ds(base, FPW)])


@jax.jit
def _face_setup(vxh, vyh, vzh, fah, fbh, fch):
    kfn = pl.kernel(
        _sc_face_setup,
        out_type=(
            jax.ShapeDtypeStruct((8, 3 * F), jnp.float32),
            jax.ShapeDtypeStruct((FCR, F), jnp.float32),
        ),
        mesh=plsc.VectorSubcoreMesh(core_axis_name="c", subcore_axis_name="s"),
        scratch_types=[
            pltpu.VMEM((V,), jnp.float32),
            pltpu.VMEM((V,), jnp.float32),
            pltpu.VMEM((V,), jnp.float32),
            pltpu.VMEM((FPW,), jnp.int32),
            pltpu.VMEM((FPW,), jnp.int32),
            pltpu.VMEM((FPW,), jnp.int32),
            pltpu.VMEM((8, FPW), jnp.float32),
            pltpu.VMEM((8, FPW), jnp.float32),
            pltpu.VMEM((8, FPW), jnp.float32),
            pltpu.VMEM((FCR, FPW), jnp.float32),
        ],
        compiler_params=pltpu.CompilerParams(needs_layout_passes=False),
    )
    return kfn(vxh, vyh, vzh, fah, fbh, fch)


def _tc_dist(pts_ref, rhs_ref, fc_ref, loss_ref, acc_ref):
    i = pl.program_id(0)
    j = pl.program_id(1)

    pts = pts_ref[...]
    col = jax.lax.broadcasted_iota(jnp.int32, (P_BLK, 8), 1)
    sq = jnp.where(col < 3, pts * pts, 0.0)
    pp = jnp.sum(sq, axis=1, keepdims=True)
    pts_aug = jnp.where(col == 3, pp, pts)

    rhsf = rhs_ref[...]
    ah = pts_aug.astype(jnp.bfloat16)
    al = (pts_aug - ah.astype(jnp.float32)).astype(jnp.bfloat16)
    bh = rhsf.astype(jnp.bfloat16)
    bl = (rhsf - bh.astype(jnp.float32)).astype(jnp.bfloat16)
    dd = lambda x, y: jax.lax.dot_general(
        x, y, (((1,), (0,)), ((), ())), preferred_element_type=jnp.float32)
    g = dd(ah, bh) + dd(ah, bl) + dd(al, bh)
    d1 = g[:, 0:F_BLK]
    d2 = g[:, F_BLK:2 * F_BLK]
    ap2 = g[:, 2 * F_BLK:3 * F_BLK]

    row = lambda r: fc_ref[r:r + 1, :]
    lab = row(R_LAB)
    lac = row(R_LAC)
    lbc = row(R_LBC)
    m = row(R_M)

    vb = lac * d1 - m * d2
    vc = lab * d2 - m * d1
    va = row(R_D) - vb - vc
    inside = jnp.minimum(va, jnp.minimum(vb, vc)) >= 0.0
    rd = row(R_RD)
    ef = row(R_DG) - ((vb * rd) * d1 + (vc * rd) * d2)
    ef = jnp.where(inside, ef, 1e30)

    d1t = d1 + d1
    ta = jnp.clip(d1 * row(R_RAB), 0.0, 1.0)
    eab = ta * (ta * lab - d1t)
    tc = jnp.clip(d2 * row(R_RAC), 0.0, 1.0)
    eac = tc * (tc * lac - (d2 + d2))
    e = (d2 - d1) + row(R_EBC)
    tb = jnp.clip(e * row(R_RBC), 0.0, 1.0)
    ebc = (lab - d1t) + tb * (tb * lbc - (e + e))

    d = ap2 + jnp.minimum(jnp.minimum(eab, eac), jnp.minimum(ebc, ef))
    dmin = jnp.maximum(jnp.min(d, axis=1, keepdims=True), 0.0)

    @pl.when((i == 0) & (j == 0))
    def _():
        loss_ref[0, 0] = 0.0

    @pl.when(j == 0)
    def _():
        acc_ref[...] = dmin

    @pl.when(j > 0)
    def _():
        acc_ref[...] = jnp.minimum(acc_ref[...], dmin)

    @pl.when(j == NF - 1)
    def _():
        loss_ref[0, 0] += jnp.sum(acc_ref[...]) * (1.0 / N)


@functools.partial(jax.jit, static_argnames=("interpret",))
def _point_mesh_loss(pts8, rhs, fcs, interpret=False):
    out = pl.pallas_call(
        _tc_dist,
        grid=(NP, NF),
        in_specs=[
            pl.BlockSpec((P_BLK, 8), lambda i, j: (i, 0)),
            pl.BlockSpec((8, 3 * F_BLK), lambda i, j: (0, j)),
            pl.BlockSpec((FCR, F_BLK), lambda i, j: (0, j)),
        ],
        out_specs=pl.BlockSpec(memory_space=pltpu.SMEM),
        out_shape=jax.ShapeDtypeStruct((1, 1), jnp.float32),
        scratch_shapes=[pltpu.VMEM((P_BLK, 1), jnp.float32)],
        compiler_params=pltpu.CompilerParams(
            dimension_semantics=("arbitrary", "arbitrary"),
        ),
        interpret=interpret,
    )(pts8, rhs, fcs)
    return out[0, 0]


def kernel(body_verts, verts, faces):
    fi = faces.astype(jnp.int32)
    rhs, fcs = _face_setup(verts[:, 0], verts[:, 1], verts[:, 2],
                           fi[:, 0], fi[:, 1], fi[:, 2])
    pts = body_verts.reshape(-1, 3)
    pad = jnp.tile(jnp.array([[0.0, 1.0, 0.0, 0.0, 0.0]], jnp.float32), (N, 1))
    pts8 = jnp.concatenate([pts, pad], axis=1)
    return _point_mesh_loss(pts8, rhs, fcs)
